# Initial kernel scaffold; baseline (speedup 1.0000x reference)
import jax
import jax.numpy as jnp
from jax.experimental import pallas as pl


def kernel(x, W):
    raise NotImplementedError("write your pallas kernel here")


# speedup vs baseline: 2.3101x; 2.3101x over previous
import functools

import jax
import jax.numpy as jnp
from jax.experimental import pallas as pl

D_MODEL_ = 768
N_EXP_ = 64
NEG_INF_ = float("-inf")


def _router_block(x_ref, wt_ref, gates_ref, idx_ref):
    logits = jnp.dot(x_ref[...], wt_ref[...],
                     preferred_element_type=jnp.float32)
    iota = jax.lax.broadcasted_iota(jnp.int32, logits.shape, 1)
    m1 = jnp.max(logits, axis=1, keepdims=True)
    a1 = jnp.min(jnp.where(logits == m1, iota, N_EXP_), axis=1,
                 keepdims=True)
    masked = jnp.where(iota == a1, NEG_INF_, logits)
    m2 = jnp.max(masked, axis=1, keepdims=True)
    a2 = jnp.min(jnp.where(masked == m2, iota, N_EXP_), axis=1,
                 keepdims=True)
    e = jnp.exp(m2 - m1)
    g1 = 1.0 / (1.0 + e)
    g2 = 1.0 - g1
    gates_ref[...] = jnp.concatenate([g1, g2], axis=1)
    idx_ref[...] = jnp.concatenate([a1, a2], axis=1)


@functools.partial(jax.jit, static_argnames=())
def kernel(x, W):
    B, S, D = x.shape
    E = W.shape[0]
    T = 2048
    n_tok = B * S
    xf = x.reshape(n_tok, D)
    wt = W.T
    grid = (n_tok // T,)
    gates, idx = pl.pallas_call(
        _router_block,
        grid=grid,
        in_specs=[
            pl.BlockSpec((T, D), lambda i: (i, 0)),
            pl.BlockSpec((D, E), lambda i: (0, 0)),
        ],
        out_specs=[
            pl.BlockSpec((T, 2), lambda i: (i, 0)),
            pl.BlockSpec((T, 2), lambda i: (i, 0)),
        ],
        out_shape=[
            jax.ShapeDtypeStruct((n_tok, 2), jnp.float32),
            jax.ShapeDtypeStruct((n_tok, 2), jnp.int32),
        ],
    )(xf, wt)
    return gates.reshape(B, S, 2), idx.reshape(B, S, 2)


# speedup vs baseline: 2.4462x; 1.0589x over previous
import functools

import jax
import jax.numpy as jnp
from jax.experimental import pallas as pl

D_MODEL_ = 768
N_EXP_ = 64
NEG_INF_ = float("-inf")
FLOAT_E_ = float(N_EXP_)


def _router_block(x_ref, wt_ref, gates_ref, idx_ref):
    logits = jnp.dot(x_ref[...], wt_ref[...],
                     preferred_element_type=jnp.float32)
    iota_f = jax.lax.broadcasted_iota(jnp.int32, logits.shape, 1).astype(
        jnp.float32)
    m1 = jnp.max(logits, axis=1, keepdims=True)
    a1f = jnp.min(jnp.where(logits == m1, iota_f, FLOAT_E_), axis=1,
                  keepdims=True)
    masked = jnp.where(iota_f == a1f, NEG_INF_, logits)
    m2 = jnp.max(masked, axis=1, keepdims=True)
    a2f = jnp.min(jnp.where(masked == m2, iota_f, FLOAT_E_), axis=1,
                  keepdims=True)
    e = jnp.exp(m2 - m1)
    g1 = 1.0 / (1.0 + e)
    g2 = 1.0 - g1
    gates_ref[...] = jnp.concatenate([g1, g2], axis=1)
    idx_ref[...] = jnp.concatenate([a1f, a2f], axis=1).astype(jnp.int32)


@functools.partial(jax.jit, static_argnames=())
def kernel(x, W):
    B, S, D = x.shape
    E = W.shape[0]
    T = 2048
    n_tok = B * S
    xf = x.reshape(n_tok, D)
    wt = W.T
    grid = (n_tok // T,)
    gates, idx = pl.pallas_call(
        _router_block,
        grid=grid,
        in_specs=[
            pl.BlockSpec((T, D), lambda i: (i, 0)),
            pl.BlockSpec((D, E), lambda i: (0, 0)),
        ],
        out_specs=[
            pl.BlockSpec((T, 2), lambda i: (i, 0)),
            pl.BlockSpec((T, 2), lambda i: (i, 0)),
        ],
        out_shape=[
            jax.ShapeDtypeStruct((n_tok, 2), jnp.float32),
            jax.ShapeDtypeStruct((n_tok, 2), jnp.int32),
        ],
    )(xf, wt)
    return gates.reshape(B, S, 2), idx.reshape(B, S, 2)


# speedup vs baseline: 2.5469x; 1.0412x over previous
import functools

import jax
import jax.numpy as jnp
from jax.experimental import pallas as pl

D_MODEL_ = 768
N_EXP_ = 64
NEG_INF_ = float("-inf")
FLOAT_E_ = float(N_EXP_)


def _router_block(x_ref, wt_ref, gates_ref, idx_ref):
    logits = jnp.dot(x_ref[...], wt_ref[...],
                     preferred_element_type=jnp.float32)
    iota_f = jax.lax.broadcasted_iota(jnp.int32, logits.shape, 1).astype(
        jnp.float32)
    m1 = jnp.max(logits, axis=1, keepdims=True)
    a1f = jnp.min(jnp.where(logits == m1, iota_f, FLOAT_E_), axis=1,
                  keepdims=True)
    masked = jnp.where(iota_f == a1f, NEG_INF_, logits)
    m2 = jnp.max(masked, axis=1, keepdims=True)
    a2f = jnp.min(jnp.where(masked == m2, iota_f, FLOAT_E_), axis=1,
                  keepdims=True)
    e = jnp.exp(m2 - m1)
    g1 = 1.0 / (1.0 + e)
    g2 = 1.0 - g1
    gates_ref[...] = jnp.concatenate([g1, g2], axis=1)
    idx_ref[...] = jnp.concatenate([a1f, a2f], axis=1).astype(jnp.int32)


@functools.partial(jax.jit, static_argnames=())
def kernel(x, W):
    B, S, D = x.shape
    E = W.shape[0]
    T = 4096
    n_tok = B * S
    xf = x.reshape(n_tok, D)
    wt = W.T
    grid = (n_tok // T,)
    gates, idx = pl.pallas_call(
        _router_block,
        grid=grid,
        in_specs=[
            pl.BlockSpec((T, D), lambda i: (i, 0)),
            pl.BlockSpec((D, E), lambda i: (0, 0)),
        ],
        out_specs=[
            pl.BlockSpec((T, 2), lambda i: (i, 0)),
            pl.BlockSpec((T, 2), lambda i: (i, 0)),
        ],
        out_shape=[
            jax.ShapeDtypeStruct((n_tok, 2), jnp.float32),
            jax.ShapeDtypeStruct((n_tok, 2), jnp.int32),
        ],
    )(xf, wt)
    return gates.reshape(B, S, 2), idx.reshape(B, S, 2)
